# Initial kernel scaffold; baseline (speedup 1.0000x reference)
import jax
import jax.numpy as jnp
from jax.experimental import pallas as pl


def kernel(node_embeddings, edge_index, W_in, b_in, W_out, b_out, W_g, b_g, W_e1, b_e1, W_e2, b_e2, W_e3, b_e3, alpha, w_mat, d_vec):
    raise NotImplementedError("write your pallas kernel here")


# speedup vs baseline: 4.3949x; 4.3949x over previous
import functools

import jax
import jax.numpy as jnp
from jax import lax
from jax.experimental import pallas as pl
from jax.experimental.pallas import tpu as pltpu
from jax.experimental.pallas import tpu_sc as plsc

N = 10000
E = 160000
DI = 64
D = 128

NP = 10240
EP = 163840
NW = 32
PER_W = EP // NW
CH = 128
NCH = PER_W // CH
RPS = NP // 16

NB = 1280
GN = NP // NB
EB = 8192
GE = EP // EB

_f32 = jnp.float32


def _mesh():
    return plsc.VectorSubcoreMesh(core_axis_name="c", subcore_axis_name="s")


def _make_dual_gather_add():
    @functools.partial(
        pl.kernel,
        out_type=jax.ShapeDtypeStruct((EP, D), _f32),
        mesh=_mesh(),
        scratch_types=[
            pltpu.VMEM((CH,), jnp.int32),
            pltpu.VMEM((CH,), jnp.int32),
            pltpu.VMEM((CH, D), _f32),
            pltpu.VMEM((CH, D), _f32),
            pltpu.VMEM((CH, D), _f32),
            pltpu.SemaphoreType.DMA,
            pltpu.SemaphoreType.DMA,
        ],
    )
    def k(s1_hbm, s2_hbm, srcg_hbm, dstg_hbm, out_hbm, i1, i2, r1, r2, g,
          sem1, sem2):
        wid = lax.axis_index("s") * 2 + lax.axis_index("c")

        def chunk(ci, _):
            base = wid * PER_W + ci * CH
            pltpu.sync_copy(srcg_hbm.at[pl.ds(base, CH)], i1)
            pltpu.sync_copy(dstg_hbm.at[pl.ds(base, CH)], i2)
            d1 = pltpu.async_copy(s1_hbm.at[i1], r1, sem1)
            d2 = pltpu.async_copy(s2_hbm.at[i2], r2, sem2)
            d1.wait()
            d2.wait()

            def row(ri, _):
                for j in range(D // 16):
                    sl = pl.ds(j * 16, 16)
                    g[ri, sl] = r1[ri, sl] + r2[ri, sl]
                return 0

            lax.fori_loop(0, CH, row, 0)
            pltpu.sync_copy(g, out_hbm.at[pl.ds(base, CH)])
            return 0

        lax.fori_loop(0, NCH, chunk, 0)

    return k


def _make_deg_scatter():
    @functools.partial(
        pl.kernel,
        out_type=jax.ShapeDtypeStruct((2, NP), _f32),
        mesh=_mesh(),
        scratch_types=[
            pltpu.VMEM((CH,), jnp.int32),
            pltpu.VMEM((CH,), _f32),
            pltpu.VMEM((RPS,), _f32),
            pltpu.VMEM_SHARED((NP,), _f32),
        ],
    )
    def k(ev_hbm, dsts_hbm, out_hbm, idx_d, ev_v, zbuf, acc):
        c = lax.axis_index("c")
        s = lax.axis_index("s")
        wid = s * 2 + c

        def zrow(i, _):
            zbuf[pl.ds(i * 16, 16)] = jnp.zeros((16,), _f32)
            return 0

        lax.fori_loop(0, RPS // 16, zrow, 0)
        pltpu.sync_copy(zbuf, acc.at[pl.ds(s * RPS, RPS)])
        plsc.subcore_barrier()

        def chunk(ci, _):
            base = wid * PER_W + ci * CH
            pltpu.sync_copy(dsts_hbm.at[pl.ds(base, CH)], idx_d)
            pltpu.sync_copy(ev_hbm.at[pl.ds(base, CH)], ev_v)
            pltpu.sync_copy(ev_v, acc.at[idx_d], add=True)
            return 0

        lax.fori_loop(0, NCH, chunk, 0)
        plsc.subcore_barrier()
        pltpu.sync_copy(acc.at[pl.ds(s * RPS, RPS)],
                        out_hbm.at[c, pl.ds(s * RPS, RPS)])

    return k


def _make_gather_scale_scatter(Dd, with_scale):
    ZR = 64
    scratch = [
        pltpu.VMEM((CH,), jnp.int32),
        pltpu.VMEM((CH,), jnp.int32),
        pltpu.VMEM((CH, Dd), _f32),
        pltpu.VMEM((CH,), _f32),
        pltpu.VMEM((ZR, Dd), _f32),
        pltpu.VMEM_SHARED((NP, Dd), _f32),
        pltpu.SemaphoreType.DMA,
    ]

    def body(table_hbm, srcg_hbm, dsts_hbm, ev_hbm, out_hbm, idx_s, idx_d,
             rows, ev_v, zbuf, acc, sem):
        c = lax.axis_index("c")
        s = lax.axis_index("s")
        wid = s * 2 + c

        def zrow(ri, _):
            for j in range(Dd // 16):
                zbuf[ri, pl.ds(j * 16, 16)] = jnp.zeros((16,), _f32)
            return 0

        lax.fori_loop(0, ZR, zrow, 0)

        def zcp(i, _):
            pltpu.sync_copy(zbuf, acc.at[pl.ds(s * RPS + i * ZR, ZR)])
            return 0

        lax.fori_loop(0, RPS // ZR, zcp, 0)
        plsc.subcore_barrier()

        def chunk(ci, _):
            base = wid * PER_W + ci * CH
            pltpu.sync_copy(srcg_hbm.at[pl.ds(base, CH)], idx_s)
            pltpu.sync_copy(dsts_hbm.at[pl.ds(base, CH)], idx_d)
            pltpu.async_copy(table_hbm.at[idx_s], rows, sem).wait()
            if with_scale:
                pltpu.sync_copy(ev_hbm.at[pl.ds(base, CH)], ev_v)

                def sgroup(gi, _):
                    evv = ev_v[pl.ds(gi * 16, 16)]
                    for ii in range(16):
                        sc = evv[ii]
                        ri = gi * 16 + ii
                        for j in range(Dd // 16):
                            sl = pl.ds(j * 16, 16)
                            rows[ri, sl] = rows[ri, sl] * sc
                    return 0

                lax.fori_loop(0, CH // 16, sgroup, 0)
            pltpu.sync_copy(rows, acc.at[idx_d], add=True)
            return 0

        lax.fori_loop(0, NCH, chunk, 0)
        plsc.subcore_barrier()
        pltpu.sync_copy(acc.at[pl.ds(s * RPS, RPS)],
                        out_hbm.at[c, pl.ds(s * RPS, RPS)])

    out_type = jax.ShapeDtypeStruct((2, NP, Dd), _f32)
    if with_scale:
        @functools.partial(pl.kernel, out_type=out_type, mesh=_mesh(),
                           scratch_types=scratch)
        def k(table_hbm, srcg_hbm, dsts_hbm, ev_hbm, out_hbm, *rest):
            body(table_hbm, srcg_hbm, dsts_hbm, ev_hbm, out_hbm, *rest)
        return k

    @functools.partial(pl.kernel, out_type=out_type, mesh=_mesh(),
                       scratch_types=scratch)
    def k2(table_hbm, srcg_hbm, dsts_hbm, out_hbm, *rest):
        body(table_hbm, srcg_hbm, dsts_hbm, None, out_hbm, *rest)
    return k2


def _nspec(r, d):
    return pl.BlockSpec((r, d), lambda i: (i, 0))


def _nspec3(r, d):
    return pl.BlockSpec((2, r, d), lambda i: (0, i, 0))


def _wspec(a, b):
    return pl.BlockSpec((a, b), lambda i: (0, 0))


def _tc_node_a(c_coef):
    with_k = c_coef != 0.0

    def body(*refs):
        if with_k:
            (cur, kprev, we1t, we1b, wg, w2m2,
             ln_o, s1_o, s2_o, xw_o, g2_o) = refs
            z = cur[...] + c_coef * kprev[...]
        else:
            cur, we1t, we1b, wg, w2m2, ln_o, s1_o, s2_o, xw_o, g2_o = refs
            z = cur[...]
        mu = jnp.mean(z, axis=1, keepdims=True)
        zc = z - mu
        var = jnp.mean(zc * zc, axis=1, keepdims=True)
        ln = zc / jnp.sqrt(var + 1e-5)
        ln_o[...] = ln
        s1_o[...] = jnp.dot(z, we1t[...], preferred_element_type=_f32)
        s2_o[...] = jnp.dot(z, we1b[...], preferred_element_type=_f32)
        xw_o[...] = jnp.dot(ln, wg[...], preferred_element_type=_f32)
        g2_o[...] = jnp.dot(z, w2m2[...], preferred_element_type=_f32)

    n_in = ([_nspec(NB, D), _nspec(NB, D)] if with_k else [_nspec(NB, D)])
    return pl.pallas_call(
        body,
        grid=(GN,),
        in_specs=n_in + [_wspec(D, D)] * 4,
        out_specs=[_nspec(NB, D)] * 5,
        out_shape=[jax.ShapeDtypeStruct((NP, D), _f32)] * 5,
    )


def _tc_edge_mlp():
    def body(g, b1, w2, b2, w3, b3, ev_o):
        h = jax.nn.silu(g[...] + b1[...])
        t = jax.nn.silu(jnp.dot(h, w2[...], preferred_element_type=_f32)
                        + b2[...])
        v = jnp.dot(t, w3[...], preferred_element_type=_f32) + b3[...]
        ev_o[...] = jax.nn.sigmoid(v)

    return pl.pallas_call(
        body,
        grid=(GE,),
        in_specs=[pl.BlockSpec((EB, D), lambda i: (i, 0)),
                  _wspec(1, D), _wspec(D, DI), _wspec(1, DI),
                  _wspec(DI, 1), _wspec(1, 1)],
        out_specs=pl.BlockSpec((EB, 1), lambda i: (i, 0)),
        out_shape=jax.ShapeDtypeStruct((EP, 1), _f32),
    )


def _tc_prep_msgs():
    def body(degp, xw, dinv_o, xwp_o):
        deg = degp[0] + degp[1] + 1.0
        dinv = lax.rsqrt(deg)
        dinv_o[...] = dinv
        xwp_o[...] = xw[...] * dinv

    return pl.pallas_call(
        body,
        grid=(GN,),
        in_specs=[_nspec3(NB, 1), _nspec(NB, D)],
        out_specs=[_nspec(NB, 1), _nspec(NB, D)],
        out_shape=[jax.ShapeDtypeStruct((NP, 1), _f32),
                   jax.ShapeDtypeStruct((NP, D), _f32)],
    )


def _tc_ode_combine():
    def body(aggp, dinv, xw, ln, g2, a2, bg, k_o):
        agg = aggp[0] + aggp[1]
        dv = dinv[...]
        gcn = dv * agg + dv * dv * xw[...] + bg[...]
        k_o[...] = a2[...] * (gcn - ln[...]) + g2[...]

    return pl.pallas_call(
        body,
        grid=(GN,),
        in_specs=[_nspec3(NB, D), _nspec(NB, 1), _nspec(NB, D),
                  _nspec(NB, D), _nspec(NB, D), _nspec(NB, 1), _wspec(1, D)],
        out_specs=_nspec(NB, D),
        out_shape=jax.ShapeDtypeStruct((NP, D), _f32),
    )


def _tc_rk4(dt):
    def body(cur, k1, k2, k3, k4, o):
        o[...] = cur[...] + (dt / 6.0) * (
            k1[...] + 2.0 * k2[...] + 2.0 * k3[...] + k4[...])

    return pl.pallas_call(
        body,
        grid=(GN,),
        in_specs=[_nspec(NB, D)] * 5,
        out_specs=_nspec(NB, D),
        out_shape=jax.ShapeDtypeStruct((NP, D), _f32),
    )


def _tc_input_mm():
    def body(emb, w, xw_o):
        xw_o[...] = jnp.dot(emb[...], w[...], preferred_element_type=_f32)

    return pl.pallas_call(
        body,
        grid=(GN,),
        in_specs=[_nspec(NB, DI), _wspec(DI, D)],
        out_specs=_nspec(NB, D),
        out_shape=jax.ShapeDtypeStruct((NP, D), _f32),
    )


def _tc_input_prep():
    def body(degp, alpha, xw, dinv_o, xwp_o, a2_o):
        deg = degp[0] + degp[1] + 1.0
        dinv = lax.rsqrt(deg)
        dinv_o[...] = dinv
        xwp_o[...] = xw[...] * dinv
        a2_o[...] = jax.nn.sigmoid(alpha[...]) * 0.5

    return pl.pallas_call(
        body,
        grid=(GN,),
        in_specs=[_nspec3(NB, 1), _nspec(NB, 1), _nspec(NB, D)],
        out_specs=[_nspec(NB, 1), _nspec(NB, D), _nspec(NB, 1)],
        out_shape=[jax.ShapeDtypeStruct((NP, 1), _f32),
                   jax.ShapeDtypeStruct((NP, D), _f32),
                   jax.ShapeDtypeStruct((NP, 1), _f32)],
    )


def _tc_input_finish():
    def body(aggp, dinv, xw, b, x_o):
        dv = dinv[...]
        x0 = dv * (aggp[0] + aggp[1]) + dv * dv * xw[...] + b[...]
        nrm = jnp.sqrt(jnp.sum(x0 * x0, axis=1, keepdims=True))
        x_o[...] = x0 / jnp.maximum(nrm, 1e-12)

    return pl.pallas_call(
        body,
        grid=(GN,),
        in_specs=[_nspec3(NB, D), _nspec(NB, 1), _nspec(NB, D), _wspec(1, D)],
        out_specs=_nspec(NB, D),
        out_shape=jax.ShapeDtypeStruct((NP, D), _f32),
    )


def _tc_output_mm():
    def body(sol, w, dinv, xw_o, xwp_o):
        sl = jax.nn.silu(sol[...])
        xw = jnp.dot(sl, w[...], preferred_element_type=_f32)
        xw_o[...] = xw
        xwp_o[...] = jnp.concatenate(
            [xw * dinv[...], jnp.zeros((NB, D - DI), _f32)], axis=1)

    return pl.pallas_call(
        body,
        grid=(GN,),
        in_specs=[_nspec(NB, D), _wspec(D, DI), _nspec(NB, 1)],
        out_specs=[_nspec(NB, DI), _nspec(NB, D)],
        out_shape=[jax.ShapeDtypeStruct((NP, DI), _f32),
                   jax.ShapeDtypeStruct((NP, D), _f32)],
    )


def _tc_output_finish():
    def body(aggp, dinv, xw, b, o):
        dv = dinv[...]
        agg = (aggp[0] + aggp[1])[:, :DI]
        o[...] = dv * agg + dv * dv * xw[...] + b[...]

    return pl.pallas_call(
        body,
        grid=(GN,),
        in_specs=[_nspec3(NB, D), _nspec(NB, 1), _nspec(NB, DI),
                  _wspec(1, DI)],
        out_specs=_nspec(NB, DI),
        out_shape=jax.ShapeDtypeStruct((NP, DI), _f32),
    )


def kernel(node_embeddings, edge_index, W_in, b_in, W_out, b_out, W_g, b_g,
           W_e1, b_e1, W_e2, b_e2, W_e3, b_e3, alpha, w_mat, d_vec):
    src = edge_index[0]
    dst = edge_index[1]
    epad = EP - E
    zpad = jnp.zeros((epad,), jnp.int32)
    srcg = jnp.concatenate([src, zpad])
    dstg = jnp.concatenate([dst, zpad])
    dsts = jnp.concatenate([dst, jnp.full((epad,), N, jnp.int32)])

    emb_p = jnp.pad(node_embeddings, ((0, NP - N), (0, 0)))
    alpha_p = jnp.pad(alpha, (0, NP - N)).reshape(NP, 1)
    ones_e = jnp.ones((EP,), _f32)

    b_in_r = b_in.reshape(1, D)
    b_out_r = b_out.reshape(1, DI)
    b_g_r = b_g.reshape(1, D)
    b_e1_r = b_e1.reshape(1, D)
    b_e2_r = b_e2.reshape(1, DI)
    b_e3_r = b_e3.reshape(1, 1)
    We1t = W_e1[:D]
    We1b = W_e1[D:]
    dcl = jnp.clip(d_vec, 0.0, 1.0)
    w2m2 = (w_mat * dcl) @ w_mat.T - 2.0 * jnp.eye(D, dtype=_f32)

    sc_gadd = _make_dual_gather_add()
    sc_deg = _make_deg_scatter()
    sc_msg = _make_gather_scale_scatter(D, True)
    sc_agg128 = _make_gather_scale_scatter(D, False)
    tc_a0 = _tc_node_a(0.0)
    tc_a_half = _tc_node_a(0.25)
    tc_a_full = _tc_node_a(0.5)
    tc_mlp = _tc_edge_mlp()
    tc_prep = _tc_prep_msgs()
    tc_ode = _tc_ode_combine()
    tc_rk4 = _tc_rk4(0.5)
    tc_in_mm = _tc_input_mm()
    tc_in_prep = _tc_input_prep()
    tc_in_fin = _tc_input_finish()
    tc_out_mm = _tc_output_mm()
    tc_out_fin = _tc_output_finish()

    xw_in = tc_in_mm(emb_p, W_in)
    deg0p = sc_deg(ones_e, dsts).reshape(2, NP, 1)
    dinv0, xwp_in, a2 = tc_in_prep(deg0p, alpha_p, xw_in)
    agg_in = sc_agg128(xwp_in, srcg, dsts)
    x = tc_in_fin(agg_in, dinv0, xw_in, b_in_r)

    def f(cur, kprev, stage):
        if stage == 0:
            ln, s1, s2, xw, g2 = tc_a0(cur, We1t, We1b, W_g, w2m2)
        elif stage in (1, 2):
            ln, s1, s2, xw, g2 = tc_a_half(cur, kprev, We1t, We1b, W_g, w2m2)
        else:
            ln, s1, s2, xw, g2 = tc_a_full(cur, kprev, We1t, We1b, W_g, w2m2)
        g = sc_gadd(s1, s2, srcg, dstg)
        ev = tc_mlp(g, b_e1_r, W_e2, b_e2_r, W_e3, b_e3_r).reshape(EP)
        degp = sc_deg(ev, dsts).reshape(2, NP, 1)
        dinv, xwp = tc_prep(degp, xw)
        agg = sc_msg(xwp, srcg, dsts, ev)
        return tc_ode(agg, dinv, xw, ln, g2, a2, b_g_r)

    sols = []
    cur = x
    for _ in range(2):
        k1 = f(cur, None, 0)
        k2 = f(cur, k1, 1)
        k3 = f(cur, k2, 2)
        k4 = f(cur, k3, 3)
        cur = tc_rk4(cur, k1, k2, k3, k4)
        sols.append(cur)

    outs = []
    for sol in sols:
        xw_o, xwp_o = tc_out_mm(sol, W_out, dinv0)
        agg_o = sc_agg128(xwp_o, srcg, dsts)
        outs.append(tc_out_fin(agg_o, dinv0, xw_o, b_out_r)[:N])
    return (jnp.stack(outs, axis=0), outs[-1])
